# Initial kernel scaffold; baseline (speedup 1.0000x reference)
import jax
import jax.numpy as jnp
from jax.experimental import pallas as pl


def kernel(x_names, x_types, x_behaviors, edge_index, edge_weight, batch, llm_features, name_emb, type_emb, llm_proj_W, llm_proj_b, conv1_W, conv1_b, conv2_W, conv2_b, cls_W, cls_b):
    raise NotImplementedError("write your pallas kernel here")


# speedup vs baseline: 4.8942x; 4.8942x over previous
import functools

import jax
import jax.numpy as jnp
from jax import lax
from jax.experimental import pallas as pl
from jax.experimental.pallas import tpu as pltpu
from jax.experimental.pallas import tpu_sc as plsc

N = 50000
E = 800000
G = 64
H = 64
HH = 32

NPAD = 50176
ROWS_W = 1568
EMB_K = 112
EMB_CH = ROWS_W // EMB_K

EPAD = 819200
DEG_E_W = EPAD // 32
MSG_E_T = EPAD // 16
CK = 128
DEG_CH = DEG_E_W // CK
MSG_CH = MSG_E_T // CK
ZROWS = 51200 // 16

_mesh = plsc.VectorSubcoreMesh(core_axis_name="c", subcore_axis_name="s")


def _bcast_lane(v, i):
    idx = jnp.full((16, 1), i, jnp.int32)
    dn = lax.GatherDimensionNumbers(
        offset_dims=(), collapsed_slice_dims=(0,), start_index_map=(0,))
    return lax.gather(v, idx, dn, (1,),
                      mode=lax.GatherScatterMode.PROMISE_IN_BOUNDS)


def _zero_vmem_1d(ref, n):
    def body(i, _):
        ref[pl.ds(i * 16, 16)] = jnp.zeros((16,), jnp.float32)
        return None
    lax.fori_loop(0, n // 16, body, None)


def _zero_vmem_2d(ref, rows, cols):
    def body(i, _):
        def inner(j, _):
            ref[i, pl.ds(j * 16, 16)] = jnp.zeros((16,), jnp.float32)
            return None
        lax.fori_loop(0, cols // 16, inner, None)
        return None
    lax.fori_loop(0, rows, body, None)


@functools.partial(
    pl.kernel,
    out_type=(
        jax.ShapeDtypeStruct((NPAD, 64), jnp.float32),
        jax.ShapeDtypeStruct((NPAD, 16), jnp.float32),
        jax.ShapeDtypeStruct((2, 51200), jnp.float32),
    ),
    mesh=_mesh,
    scratch_types=[
        pltpu.VMEM((EMB_K,), jnp.int32),
        pltpu.VMEM((EMB_K, 64), jnp.float32),
        pltpu.VMEM((EMB_K, 16), jnp.float32),
        pltpu.VMEM((1, CK), jnp.int32),
        pltpu.VMEM((CK,), jnp.float32),
        pltpu.VMEM((ZROWS,), jnp.float32),
        pltpu.VMEM_SHARED((51200,), jnp.float32),
        pltpu.SemaphoreType.DMA,
    ],
    compiler_params=pltpu.CompilerParams(use_tc_tiling_on_sc=False),
)
def sc_embed_deg(names_hbm, types_hbm, dst_hbm, w_hbm, name_tab, type_tab,
                 nf_out, tf_out, deg_out,
                 idx_v, nrows_v, trows_v, didx_v, wv, zv, deg_sp, sem):
    c = lax.axis_index("c")
    s = lax.axis_index("s")
    wid = s * 2 + c

    _zero_vmem_1d(zv, ZROWS)
    pltpu.sync_copy(zv, deg_sp.at[pl.ds(s * ZROWS, ZROWS)])
    plsc.subcore_barrier()

    def emb_chunk(k, _):
        base = wid * ROWS_W + k * EMB_K
        pltpu.sync_copy(names_hbm.at[pl.ds(base, EMB_K)], idx_v)
        pltpu.async_copy(name_tab.at[idx_v], nrows_v, sem).wait()
        pltpu.sync_copy(nrows_v, nf_out.at[pl.ds(base, EMB_K)])
        pltpu.sync_copy(types_hbm.at[pl.ds(base, EMB_K)], idx_v)
        pltpu.async_copy(type_tab.at[idx_v], trows_v, sem).wait()
        pltpu.sync_copy(trows_v, tf_out.at[pl.ds(base, EMB_K)])
        return None
    lax.fori_loop(0, EMB_CH, emb_chunk, None)

    def deg_chunk(k, _):
        ebase = wid * DEG_E_W + k * CK
        pltpu.sync_copy(w_hbm.at[pl.ds(ebase, CK)], wv)
        pltpu.sync_copy(dst_hbm.at[pl.ds(ebase, CK)], didx_v.at[0])
        pltpu.sync_copy(wv, deg_sp.at[didx_v.at[0]], add=True)
        return None
    lax.fori_loop(0, DEG_CH, deg_chunk, None)

    plsc.subcore_barrier()
    pltpu.sync_copy(deg_sp.at[pl.ds(s * ZROWS, ZROWS)],
                    deg_out.at[c, pl.ds(s * ZROWS, ZROWS)])


QQ = 16


@functools.partial(
    pl.kernel,
    out_type=jax.ShapeDtypeStruct((2, 51200, QQ), jnp.float32),
    mesh=_mesh,
    scratch_types=[
        pltpu.VMEM((CK,), jnp.int32),
        pltpu.VMEM((1, CK), jnp.int32),
        pltpu.VMEM((CK,), jnp.float32),
        pltpu.VMEM((CK, QQ), jnp.float32),
        pltpu.VMEM((ZROWS, QQ), jnp.float32),
        pltpu.VMEM_SHARED((51200, QQ), jnp.float32),
        pltpu.SemaphoreType.DMA,
    ],
    compiler_params=pltpu.CompilerParams(use_tc_tiling_on_sc=False),
)
def sc_edge_msg(ya_hbm, yb_hbm, src_hbm, dst_hbm, w_hbm, z_out,
                sidx_v, didx_v, wv, rows_v, zv, z_sp, sem):
    c = lax.axis_index("c")
    s = lax.axis_index("s")

    _zero_vmem_2d(zv, ZROWS, QQ)
    pltpu.sync_copy(zv, z_sp.at[pl.ds(s * ZROWS, ZROWS)])
    plsc.subcore_barrier()

    def chunk(k, _):
        ebase = s * MSG_E_T + k * CK
        pltpu.sync_copy(src_hbm.at[pl.ds(ebase, CK)], sidx_v)
        pltpu.sync_copy(dst_hbm.at[pl.ds(ebase, CK)], didx_v.at[0])
        pltpu.sync_copy(w_hbm.at[pl.ds(ebase, CK)], wv)

        @pl.when(c == 0)
        def _():
            pltpu.async_copy(ya_hbm.at[sidx_v], rows_v, sem).wait()

        @pl.when(c == 1)
        def _():
            pltpu.async_copy(yb_hbm.at[sidx_v], rows_v, sem).wait()

        def grp(g, _):
            wvec = wv[pl.ds(g * 16, 16)]
            for i in range(16):
                e = g * 16 + i
                rows_v[e, pl.ds(0, QQ)] = (rows_v[e, pl.ds(0, QQ)]
                                           * _bcast_lane(wvec, i))
            return None
        lax.fori_loop(0, CK // 16, grp, None)

        pltpu.sync_copy(rows_v, z_sp.at[didx_v.at[0]], add=True)
        return None
    lax.fori_loop(0, MSG_CH, chunk, None)

    plsc.subcore_barrier()
    pltpu.sync_copy(z_sp.at[pl.ds(s * ZROWS, ZROWS)],
                    z_out.at[c, pl.ds(s * ZROWS, ZROWS)])


R = 1000
GRID = N // R


def _tc_a_body(nf, tf, llm, beh, d0, d1, w1a, w1b, p1, w1d, rb,
               y0_o, y1_o, y2_o, y3_o, dinv_o):
    y = (jnp.dot(nf[...], w1a[...], preferred_element_type=jnp.float32)
         + jnp.dot(tf[...], w1b[...], preferred_element_type=jnp.float32)
         + jnp.dot(llm[...], p1[...], preferred_element_type=jnp.float32)
         + jnp.dot(beh[...], w1d[...], preferred_element_type=jnp.float32)
         + rb[...])
    deg = d0[...] + d1[...] + 1.0
    dinv = jnp.where(deg > 0, lax.rsqrt(jnp.where(deg > 0, deg, 1.0)), 0.0)
    yp = y * dinv
    y0_o[...] = yp[:, 0:16]
    y1_o[...] = yp[:, 16:32]
    y2_o[...] = yp[:, 32:48]
    y3_o[...] = yp[:, 48:64]
    dinv_o[...] = dinv


def _tc_b_body(z0, z1, z2, z3, y0, y1, y2, y3, dinv, w2, b1,
               o0, o1, o2, o3):
    z = jnp.concatenate([z0[...], z1[...], z2[...], z3[...]], axis=1)
    yp = jnp.concatenate([y0[...], y1[...], y2[...], y3[...]], axis=1)
    h1 = jnp.maximum(dinv[...] * (z + yp) + b1[...], 0.0)
    y2v = jnp.dot(h1, w2[...], preferred_element_type=jnp.float32)
    y2p = y2v * dinv[...]
    o0[...] = y2p[:, 0:16]
    o1[...] = y2p[:, 16:32]
    o2[...] = y2p[:, 32:48]
    o3[...] = y2p[:, 48:64]


def _tc_c_body(z0, z1, z2, z3, y0, y1, y2, y3, dinv, b2, batch, cw, cb,
               out, acc, cnt):
    i = pl.program_id(0)

    @pl.when(i == 0)
    def _():
        acc[...] = jnp.zeros((G, H), jnp.float32)
        cnt[...] = jnp.zeros((G, 1), jnp.float32)

    z = jnp.concatenate([z0[...], z1[...], z2[...], z3[...]], axis=1)
    yp = jnp.concatenate([y0[...], y1[...], y2[...], y3[...]], axis=1)
    h2 = jnp.maximum(dinv[...] * (z + yp) + b2[...], 0.0)
    seg = lax.broadcasted_iota(jnp.int32, (R, G), 1)
    onehot = (seg == batch[...]).astype(jnp.float32)
    dn = (((0,), (0,)), ((), ()))
    acc[...] += lax.dot_general(onehot, h2, dn,
                                preferred_element_type=jnp.float32)
    cnt[...] += lax.dot_general(onehot, jnp.ones((R, 1), jnp.float32), dn,
                                preferred_element_type=jnp.float32)

    @pl.when(i == GRID - 1)
    def _():
        pooled = acc[...] / jnp.maximum(cnt[...], 1.0)
        out[...] = (jnp.dot(pooled, cw[...], preferred_element_type=jnp.float32)
                    + cb[...])


def _row_spec(cols):
    return pl.BlockSpec((R, cols), lambda i: (i, 0))


def _full_spec(r, c):
    return pl.BlockSpec((r, c), lambda i: (0, 0))


def kernel(x_names, x_types, x_behaviors, edge_index, edge_weight, batch,
           llm_features, name_emb, type_emb, llm_proj_W, llm_proj_b,
           conv1_W, conv1_b, conv2_W, conv2_b, cls_W, cls_b):
    f32 = jnp.float32
    names_p = jnp.pad(x_names.astype(jnp.int32), (0, NPAD - N))
    types_p = jnp.pad(x_types.astype(jnp.int32), (0, NPAD - N))
    src_p = jnp.pad(edge_index[0].astype(jnp.int32), (0, EPAD - E))
    dst_p = jnp.pad(edge_index[1].astype(jnp.int32), (0, EPAD - E))
    w_p = jnp.pad(edge_weight.astype(f32), (0, EPAD - E))

    w1a = conv1_W[0:64]
    w1b = conv1_W[64:80]
    w1c = conv1_W[80:112]
    w1d = conv1_W[112:128]
    p1 = llm_proj_W @ w1c
    rb = (llm_proj_b @ w1c).reshape(1, H)

    nf_p, tf_p, deg2 = sc_embed_deg(names_p, types_p, dst_p, w_p,
                                    name_emb.astype(f32), type_emb.astype(f32))
    nf = nf_p[:N]
    tf = tf_p[:N]
    d0 = deg2[0, :N].reshape(N, 1)
    d1 = deg2[1, :N].reshape(N, 1)

    qspec = _row_spec(QQ)
    qshape = jax.ShapeDtypeStruct((N, QQ), f32)

    y1q = pl.pallas_call(
        _tc_a_body,
        grid=(GRID,),
        in_specs=[
            _row_spec(64), _row_spec(16), _row_spec(768), _row_spec(16),
            _row_spec(1), _row_spec(1),
            _full_spec(64, H), _full_spec(16, H), _full_spec(768, H),
            _full_spec(16, H), _full_spec(1, H),
        ],
        out_specs=[qspec, qspec, qspec, qspec, _row_spec(1)],
        out_shape=[qshape, qshape, qshape, qshape,
                   jax.ShapeDtypeStruct((N, 1), f32)],
    )(nf, tf, llm_features.astype(f32), x_behaviors.astype(f32), d0, d1,
      w1a, w1b, p1, w1d, rb)
    dinv = y1q[4]

    za = sc_edge_msg(y1q[0], y1q[1], src_p, dst_p, w_p)
    zb = sc_edge_msg(y1q[2], y1q[3], src_p, dst_p, w_p)
    z1 = (za[0, :N], za[1, :N], zb[0, :N], zb[1, :N])

    y2q = pl.pallas_call(
        _tc_b_body,
        grid=(GRID,),
        in_specs=[qspec] * 8 + [_row_spec(1), _full_spec(H, H),
                                _full_spec(1, H)],
        out_specs=[qspec, qspec, qspec, qspec],
        out_shape=[qshape, qshape, qshape, qshape],
    )(*z1, y1q[0], y1q[1], y1q[2], y1q[3], dinv, conv2_W.astype(f32),
      conv1_b.reshape(1, H).astype(f32))

    za = sc_edge_msg(y2q[0], y2q[1], src_p, dst_p, w_p)
    zb = sc_edge_msg(y2q[2], y2q[3], src_p, dst_p, w_p)
    z2 = (za[0, :N], za[1, :N], zb[0, :N], zb[1, :N])

    logits = pl.pallas_call(
        _tc_c_body,
        grid=(GRID,),
        in_specs=[qspec] * 8 + [
            _row_spec(1), _full_spec(1, H), _row_spec(1),
            _full_spec(H, 2), _full_spec(1, 2),
        ],
        out_specs=pl.BlockSpec((G, 2), lambda i: (0, 0)),
        out_shape=jax.ShapeDtypeStruct((G, 2), f32),
        scratch_shapes=[
            pltpu.VMEM((G, H), f32),
            pltpu.VMEM((G, 1), f32),
        ],
        compiler_params=pltpu.CompilerParams(
            dimension_semantics=("arbitrary",)),
    )(*z2, y2q[0], y2q[1], y2q[2], y2q[3], dinv,
      conv2_b.reshape(1, H).astype(f32),
      batch.astype(jnp.int32).reshape(N, 1), cls_W.astype(f32),
      cls_b.reshape(1, 2).astype(f32))

    return logits


# speedup vs baseline: 7.6794x; 1.5691x over previous
import functools

import jax
import jax.numpy as jnp
from jax import lax
from jax.experimental import pallas as pl
from jax.experimental.pallas import tpu as pltpu
from jax.experimental.pallas import tpu_sc as plsc

N = 50000
E = 800000
G = 64
H = 64
HH = 32

NPAD = 50176
ROWS_W = 1568
EMB_K = 112
EMB_CH = ROWS_W // EMB_K

EPAD = 819200
DEG_E_W = EPAD // 32
MSG_E_T = EPAD // 16
CK = 128
DEG_CH = DEG_E_W // CK
MSG_CH = MSG_E_T // CK
ZROWS = 51200 // 16

_mesh = plsc.VectorSubcoreMesh(core_axis_name="c", subcore_axis_name="s")


def _bcast_lane(v, i):
    idx = jnp.full((16, 1), i, jnp.int32)
    dn = lax.GatherDimensionNumbers(
        offset_dims=(), collapsed_slice_dims=(0,), start_index_map=(0,))
    return lax.gather(v, idx, dn, (1,),
                      mode=lax.GatherScatterMode.PROMISE_IN_BOUNDS)


def _zero_vmem_1d(ref, n):
    def body(i, _):
        ref[pl.ds(i * 16, 16)] = jnp.zeros((16,), jnp.float32)
        return None
    lax.fori_loop(0, n // 16, body, None)


def _zero_vmem_2d(ref, rows, cols):
    def body(i, _):
        def inner(j, _):
            ref[i, pl.ds(j * 16, 16)] = jnp.zeros((16,), jnp.float32)
            return None
        lax.fori_loop(0, cols // 16, inner, None)
        return None
    lax.fori_loop(0, rows, body, None)


@functools.partial(
    pl.kernel,
    out_type=(
        jax.ShapeDtypeStruct((NPAD, 64), jnp.float32),
        jax.ShapeDtypeStruct((NPAD, 16), jnp.float32),
        jax.ShapeDtypeStruct((2, 51200), jnp.float32),
    ),
    mesh=_mesh,
    scratch_types=[
        pltpu.VMEM((EMB_K,), jnp.int32),
        pltpu.VMEM((EMB_K, 64), jnp.float32),
        pltpu.VMEM((EMB_K, 16), jnp.float32),
        pltpu.VMEM((1, CK), jnp.int32),
        pltpu.VMEM((CK,), jnp.float32),
        pltpu.VMEM((ZROWS,), jnp.float32),
        pltpu.VMEM_SHARED((51200,), jnp.float32),
        pltpu.SemaphoreType.DMA,
    ],
    compiler_params=pltpu.CompilerParams(use_tc_tiling_on_sc=False),
)
def sc_embed_deg(names_hbm, types_hbm, dst_hbm, w_hbm, name_tab, type_tab,
                 nf_out, tf_out, deg_out,
                 idx_v, nrows_v, trows_v, didx_v, wv, zv, deg_sp, sem):
    c = lax.axis_index("c")
    s = lax.axis_index("s")
    wid = s * 2 + c

    _zero_vmem_1d(zv, ZROWS)
    pltpu.sync_copy(zv, deg_sp.at[pl.ds(s * ZROWS, ZROWS)])
    plsc.subcore_barrier()

    def emb_chunk(k, _):
        base = wid * ROWS_W + k * EMB_K
        pltpu.sync_copy(names_hbm.at[pl.ds(base, EMB_K)], idx_v)
        pltpu.async_copy(name_tab.at[idx_v], nrows_v, sem).wait()
        pltpu.sync_copy(nrows_v, nf_out.at[pl.ds(base, EMB_K)])
        pltpu.sync_copy(types_hbm.at[pl.ds(base, EMB_K)], idx_v)
        pltpu.async_copy(type_tab.at[idx_v], trows_v, sem).wait()
        pltpu.sync_copy(trows_v, tf_out.at[pl.ds(base, EMB_K)])
        return None
    lax.fori_loop(0, EMB_CH, emb_chunk, None)

    def deg_chunk(k, _):
        ebase = wid * DEG_E_W + k * CK
        pltpu.sync_copy(w_hbm.at[pl.ds(ebase, CK)], wv)
        pltpu.sync_copy(dst_hbm.at[pl.ds(ebase, CK)], didx_v.at[0])
        pltpu.sync_copy(wv, deg_sp.at[didx_v.at[0]], add=True)
        return None
    lax.fori_loop(0, DEG_CH, deg_chunk, None)

    plsc.subcore_barrier()
    pltpu.sync_copy(deg_sp.at[pl.ds(s * ZROWS, ZROWS)],
                    deg_out.at[c, pl.ds(s * ZROWS, ZROWS)])


QQ = 16


SUP = 1024
NSUB = SUP // CK
NSUP = MSG_E_T // SUP


@functools.partial(
    pl.kernel,
    out_type=jax.ShapeDtypeStruct((2, 51200, QQ), jnp.float32),
    mesh=_mesh,
    scratch_types=[
        pltpu.VMEM((SUP,), jnp.int32),
        pltpu.VMEM((NSUB, CK), jnp.int32),
        pltpu.VMEM((SUP,), jnp.float32),
        pltpu.VMEM((2, CK, QQ), jnp.float32),
        pltpu.VMEM((ZROWS, QQ), jnp.float32),
        pltpu.VMEM_SHARED((51200, QQ), jnp.float32),
        pltpu.SemaphoreType.DMA,
        pltpu.SemaphoreType.DMA,
    ],
    compiler_params=pltpu.CompilerParams(use_tc_tiling_on_sc=False),
)
def sc_edge_msg(ya_hbm, yb_hbm, src_hbm, dst_hbm, w_hbm, z_out,
                sidx_v, didx_v, wv, rows_v, zv, z_sp, sem0, sem1):
    c = lax.axis_index("c")
    s = lax.axis_index("s")
    sems = (sem0, sem1)

    _zero_vmem_2d(zv, ZROWS, QQ)
    pltpu.sync_copy(zv, z_sp.at[pl.ds(s * ZROWS, ZROWS)])
    plsc.subcore_barrier()

    def gather(j):
        p = j % 2
        idx = sidx_v.at[pl.ds(j * CK, CK)]

        @pl.when(c == 0)
        def _():
            pltpu.async_copy(ya_hbm.at[idx], rows_v.at[p], sems[p])

        @pl.when(c == 1)
        def _():
            pltpu.async_copy(yb_hbm.at[idx], rows_v.at[p], sems[p])

    def drain(j):
        p = j % 2
        pltpu.make_async_copy(ya_hbm.at[sidx_v.at[pl.ds(0, CK)]],
                              rows_v.at[p], sems[p]).wait()

    def chunk(k, _):
        ebase = s * MSG_E_T + k * SUP
        pltpu.sync_copy(src_hbm.at[pl.ds(ebase, SUP)], sidx_v)
        pltpu.sync_copy(w_hbm.at[pl.ds(ebase, SUP)], wv)
        for j in range(NSUB):
            pltpu.sync_copy(dst_hbm.at[pl.ds(ebase + j * CK, CK)],
                            didx_v.at[j])
        gather(0)
        for j in range(NSUB):
            if j + 1 < NSUB:
                gather(j + 1)
            drain(j)
            p = j % 2

            def grp(g, _, p=p, j=j):
                wvec = wv[pl.ds(j * CK + g * 16, 16)]
                for i in range(16):
                    e = g * 16 + i
                    rows_v[p, e, pl.ds(0, QQ)] = (
                        rows_v[p, e, pl.ds(0, QQ)] * _bcast_lane(wvec, i))
                return None
            lax.fori_loop(0, CK // 16, grp, None)

            pltpu.sync_copy(rows_v.at[p], z_sp.at[didx_v.at[j]], add=True)
        return None
    lax.fori_loop(0, NSUP, chunk, None)

    plsc.subcore_barrier()
    pltpu.sync_copy(z_sp.at[pl.ds(s * ZROWS, ZROWS)],
                    z_out.at[c, pl.ds(s * ZROWS, ZROWS)])


R = 1000
GRID = N // R


def _tc_a_body(nf, tf, llm, beh, d0, d1, w1a, w1b, p1, w1d, rb,
               y0_o, y1_o, y2_o, y3_o, dinv_o):
    y = (jnp.dot(nf[...], w1a[...], preferred_element_type=jnp.float32)
         + jnp.dot(tf[...], w1b[...], preferred_element_type=jnp.float32)
         + jnp.dot(llm[...], p1[...], preferred_element_type=jnp.float32)
         + jnp.dot(beh[...], w1d[...], preferred_element_type=jnp.float32)
         + rb[...])
    deg = d0[...] + d1[...] + 1.0
    dinv = jnp.where(deg > 0, lax.rsqrt(jnp.where(deg > 0, deg, 1.0)), 0.0)
    yp = y * dinv
    y0_o[...] = yp[:, 0:16]
    y1_o[...] = yp[:, 16:32]
    y2_o[...] = yp[:, 32:48]
    y3_o[...] = yp[:, 48:64]
    dinv_o[...] = dinv


def _tc_b_body(z0, z1, z2, z3, y0, y1, y2, y3, dinv, w2, b1,
               o0, o1, o2, o3):
    z = jnp.concatenate([z0[...], z1[...], z2[...], z3[...]], axis=1)
    yp = jnp.concatenate([y0[...], y1[...], y2[...], y3[...]], axis=1)
    h1 = jnp.maximum(dinv[...] * (z + yp) + b1[...], 0.0)
    y2v = jnp.dot(h1, w2[...], preferred_element_type=jnp.float32)
    y2p = y2v * dinv[...]
    o0[...] = y2p[:, 0:16]
    o1[...] = y2p[:, 16:32]
    o2[...] = y2p[:, 32:48]
    o3[...] = y2p[:, 48:64]


def _tc_c_body(z0, z1, z2, z3, y0, y1, y2, y3, dinv, b2, batch, cw, cb,
               out, acc, cnt):
    i = pl.program_id(0)

    @pl.when(i == 0)
    def _():
        acc[...] = jnp.zeros((G, H), jnp.float32)
        cnt[...] = jnp.zeros((G, 1), jnp.float32)

    z = jnp.concatenate([z0[...], z1[...], z2[...], z3[...]], axis=1)
    yp = jnp.concatenate([y0[...], y1[...], y2[...], y3[...]], axis=1)
    h2 = jnp.maximum(dinv[...] * (z + yp) + b2[...], 0.0)
    seg = lax.broadcasted_iota(jnp.int32, (R, G), 1)
    onehot = (seg == batch[...]).astype(jnp.float32)
    dn = (((0,), (0,)), ((), ()))
    acc[...] += lax.dot_general(onehot, h2, dn,
                                preferred_element_type=jnp.float32)
    cnt[...] += lax.dot_general(onehot, jnp.ones((R, 1), jnp.float32), dn,
                                preferred_element_type=jnp.float32)

    @pl.when(i == GRID - 1)
    def _():
        pooled = acc[...] / jnp.maximum(cnt[...], 1.0)
        out[...] = (jnp.dot(pooled, cw[...], preferred_element_type=jnp.float32)
                    + cb[...])


def _row_spec(cols):
    return pl.BlockSpec((R, cols), lambda i: (i, 0))


def _full_spec(r, c):
    return pl.BlockSpec((r, c), lambda i: (0, 0))


def kernel(x_names, x_types, x_behaviors, edge_index, edge_weight, batch,
           llm_features, name_emb, type_emb, llm_proj_W, llm_proj_b,
           conv1_W, conv1_b, conv2_W, conv2_b, cls_W, cls_b):
    f32 = jnp.float32
    names_p = jnp.pad(x_names.astype(jnp.int32), (0, NPAD - N))
    types_p = jnp.pad(x_types.astype(jnp.int32), (0, NPAD - N))
    src_p = jnp.pad(edge_index[0].astype(jnp.int32), (0, EPAD - E))
    dst_p = jnp.pad(edge_index[1].astype(jnp.int32), (0, EPAD - E))
    w_p = jnp.pad(edge_weight.astype(f32), (0, EPAD - E))

    w1a = conv1_W[0:64]
    w1b = conv1_W[64:80]
    w1c = conv1_W[80:112]
    w1d = conv1_W[112:128]
    p1 = llm_proj_W @ w1c
    rb = (llm_proj_b @ w1c).reshape(1, H)

    nf_p, tf_p, deg2 = sc_embed_deg(names_p, types_p, dst_p, w_p,
                                    name_emb.astype(f32), type_emb.astype(f32))
    nf = nf_p[:N]
    tf = tf_p[:N]
    d0 = deg2[0, :N].reshape(N, 1)
    d1 = deg2[1, :N].reshape(N, 1)

    qspec = _row_spec(QQ)
    qshape = jax.ShapeDtypeStruct((N, QQ), f32)

    y1q = pl.pallas_call(
        _tc_a_body,
        grid=(GRID,),
        in_specs=[
            _row_spec(64), _row_spec(16), _row_spec(768), _row_spec(16),
            _row_spec(1), _row_spec(1),
            _full_spec(64, H), _full_spec(16, H), _full_spec(768, H),
            _full_spec(16, H), _full_spec(1, H),
        ],
        out_specs=[qspec, qspec, qspec, qspec, _row_spec(1)],
        out_shape=[qshape, qshape, qshape, qshape,
                   jax.ShapeDtypeStruct((N, 1), f32)],
    )(nf, tf, llm_features.astype(f32), x_behaviors.astype(f32), d0, d1,
      w1a, w1b, p1, w1d, rb)
    dinv = y1q[4]

    za = sc_edge_msg(y1q[0], y1q[1], src_p, dst_p, w_p)
    zb = sc_edge_msg(y1q[2], y1q[3], src_p, dst_p, w_p)
    z1 = (za[0, :N], za[1, :N], zb[0, :N], zb[1, :N])

    y2q = pl.pallas_call(
        _tc_b_body,
        grid=(GRID,),
        in_specs=[qspec] * 8 + [_row_spec(1), _full_spec(H, H),
                                _full_spec(1, H)],
        out_specs=[qspec, qspec, qspec, qspec],
        out_shape=[qshape, qshape, qshape, qshape],
    )(*z1, y1q[0], y1q[1], y1q[2], y1q[3], dinv, conv2_W.astype(f32),
      conv1_b.reshape(1, H).astype(f32))

    za = sc_edge_msg(y2q[0], y2q[1], src_p, dst_p, w_p)
    zb = sc_edge_msg(y2q[2], y2q[3], src_p, dst_p, w_p)
    z2 = (za[0, :N], za[1, :N], zb[0, :N], zb[1, :N])

    logits = pl.pallas_call(
        _tc_c_body,
        grid=(GRID,),
        in_specs=[qspec] * 8 + [
            _row_spec(1), _full_spec(1, H), _row_spec(1),
            _full_spec(H, 2), _full_spec(1, 2),
        ],
        out_specs=pl.BlockSpec((G, 2), lambda i: (0, 0)),
        out_shape=jax.ShapeDtypeStruct((G, 2), f32),
        scratch_shapes=[
            pltpu.VMEM((G, H), f32),
            pltpu.VMEM((G, 1), f32),
        ],
        compiler_params=pltpu.CompilerParams(
            dimension_semantics=("arbitrary",)),
    )(*z2, y2q[0], y2q[1], y2q[2], y2q[3], dinv,
      conv2_b.reshape(1, H).astype(f32),
      batch.astype(jnp.int32).reshape(N, 1), cls_W.astype(f32),
      cls_b.reshape(1, 2).astype(f32))

    return logits
